# Initial kernel scaffold; baseline (speedup 1.0000x reference)
import jax
import jax.numpy as jnp
from jax.experimental import pallas as pl


def kernel(x, edge_index, W1, b1, W2, b2):
    raise NotImplementedError("write your pallas kernel here")


# speedup vs baseline: 51.1991x; 51.1991x over previous
import functools

import jax
import jax.numpy as jnp
from jax.experimental import pallas as pl
from jax.experimental.pallas import tpu as pltpu
from jax.experimental.pallas import tpu_sc as plsc

N = 10000
E = 320000
D = 128
H1 = 16
H2 = 2

NPAD = 10240
NC, NS = 2, 16
NW = NC * NS
NROWS = 2560
EPAD = NROWS * 128
RPT = NROWS // NW
NODE_ROWS_PER_TILE = NPAD // NS

_MESH = plsc.VectorSubcoreMesh(core_axis_name="c", subcore_axis_name="s")

_PARTIAL = jax.ShapeDtypeStruct((NC, NPAD, H1), jnp.float32)


@functools.partial(
    pl.kernel,
    out_type=_PARTIAL,
    mesh=_MESH,
    scratch_types=[
        pltpu.VMEM((RPT, 128), jnp.int32),
        pltpu.VMEM((128, H1), jnp.float32),
        pltpu.VMEM_SHARED((NPAD, H1), jnp.float32),
    ],
)
def _sc_degree(dst_hbm, out_hbm, dsti, buf, accum):
    c = jax.lax.axis_index("c")
    s = jax.lax.axis_index("s")
    wid = s * NC + c

    @pl.loop(0, 128)
    def _(i):
        buf[i, :] = jnp.zeros((H1,), jnp.float32)

    @pl.loop(0, NODE_ROWS_PER_TILE // 128)
    def _(q):
        pltpu.sync_copy(buf, accum.at[pl.ds(s * NODE_ROWS_PER_TILE + q * 128, 128)])

    pltpu.sync_copy(dst_hbm.at[pl.ds(wid * RPT, RPT)], dsti)

    @pl.loop(0, 128)
    def _(i):
        buf[i, :] = jnp.full((H1,), 1.0, jnp.float32)

    plsc.subcore_barrier()

    @pl.loop(0, RPT)
    def _(j):
        pltpu.sync_copy(buf, accum.at[dsti.at[j]], add=True)

    plsc.subcore_barrier()
    pltpu.sync_copy(
        accum.at[pl.ds(s * NODE_ROWS_PER_TILE, NODE_ROWS_PER_TILE)],
        out_hbm.at[c, pl.ds(s * NODE_ROWS_PER_TILE, NODE_ROWS_PER_TILE)],
    )


@functools.partial(
    pl.kernel,
    out_type=_PARTIAL,
    mesh=_MESH,
    scratch_types=[
        pltpu.VMEM((RPT, 128), jnp.int32),
        pltpu.VMEM((RPT, 128), jnp.int32),
        pltpu.VMEM((128, H1), jnp.float32),
        pltpu.VMEM_SHARED((NPAD, H1), jnp.float32),
        pltpu.VMEM_SHARED((NPAD, H1), jnp.float32),
        pltpu.SemaphoreType.DMA,
    ],
)
def _sc_aggregate(src_hbm, dst_hbm, g_hbm, out_hbm, srci, dsti, rows, accum, gsh, sem):
    c = jax.lax.axis_index("c")
    s = jax.lax.axis_index("s")
    wid = s * NC + c

    @pl.loop(0, 128)
    def _(i):
        rows[i, :] = jnp.zeros((H1,), jnp.float32)

    @pl.loop(0, NODE_ROWS_PER_TILE // 128)
    def _(q):
        pltpu.sync_copy(rows, accum.at[pl.ds(s * NODE_ROWS_PER_TILE + q * 128, 128)])

    pltpu.sync_copy(
        g_hbm.at[pl.ds(s * NODE_ROWS_PER_TILE, NODE_ROWS_PER_TILE)],
        gsh.at[pl.ds(s * NODE_ROWS_PER_TILE, NODE_ROWS_PER_TILE)],
    )
    pltpu.sync_copy(src_hbm.at[pl.ds(wid * RPT, RPT)], srci)
    pltpu.sync_copy(dst_hbm.at[pl.ds(wid * RPT, RPT)], dsti)
    plsc.subcore_barrier()

    @pl.loop(0, RPT)
    def _(j):
        pltpu.async_copy(gsh.at[srci.at[j]], rows, sem).wait()
        pltpu.sync_copy(rows, accum.at[dsti.at[j]], add=True)

    plsc.subcore_barrier()
    pltpu.sync_copy(
        accum.at[pl.ds(s * NODE_ROWS_PER_TILE, NODE_ROWS_PER_TILE)],
        out_hbm.at[c, pl.ds(s * NODE_ROWS_PER_TILE, NODE_ROWS_PER_TILE)],
    )


def _tc_matmul1(x_pad, W1):
    def body(x_ref, w_ref, o_ref):
        o_ref[...] = jnp.dot(x_ref[...], w_ref[...],
                             preferred_element_type=jnp.float32)

    return pl.pallas_call(
        body,
        grid=(NPAD // 1280,),
        in_specs=[
            pl.BlockSpec((1280, D), lambda i: (i, 0)),
            pl.BlockSpec((D, H1), lambda i: (0, 0)),
        ],
        out_specs=pl.BlockSpec((1280, H1), lambda i: (i, 0)),
        out_shape=jax.ShapeDtypeStruct((NPAD, H1), jnp.float32),
    )(x_pad, W1)


def _tc_scale(cnt_p, h1):
    def body(cnt_ref, h_ref, dinv_ref, g_ref):
        cnt = cnt_ref[0] + cnt_ref[1]
        dinv = jax.lax.rsqrt(1.0 + cnt)
        dinv_ref[...] = dinv
        g_ref[...] = dinv * h_ref[...]

    return pl.pallas_call(
        body,
        grid=(NPAD // 1280,),
        in_specs=[
            pl.BlockSpec((NC, 1280, H1), lambda i: (0, i, 0)),
            pl.BlockSpec((1280, H1), lambda i: (i, 0)),
        ],
        out_specs=[
            pl.BlockSpec((1280, H1), lambda i: (i, 0)),
            pl.BlockSpec((1280, H1), lambda i: (i, 0)),
        ],
        out_shape=[
            jax.ShapeDtypeStruct((NPAD, H1), jnp.float32),
            jax.ShapeDtypeStruct((NPAD, H1), jnp.float32),
        ],
    )(cnt_p, h1)


def _tc_relu_scale(s1_p, g1, dinv16, b1r):
    def body(s_ref, g_ref, dinv_ref, b_ref, o_ref):
        agg = s_ref[0] + s_ref[1] + g_ref[...]
        z = dinv_ref[...] * agg + b_ref[...]
        h = jnp.maximum(z, 0.0)
        o_ref[...] = dinv_ref[...] * h

    return pl.pallas_call(
        body,
        grid=(NPAD // 1280,),
        in_specs=[
            pl.BlockSpec((NC, 1280, H1), lambda i: (0, i, 0)),
            pl.BlockSpec((1280, H1), lambda i: (i, 0)),
            pl.BlockSpec((1280, H1), lambda i: (i, 0)),
            pl.BlockSpec((1, H1), lambda i: (0, 0)),
        ],
        out_specs=pl.BlockSpec((1280, H1), lambda i: (i, 0)),
        out_shape=jax.ShapeDtypeStruct((NPAD, H1), jnp.float32),
    )(s1_p, g1, dinv16, b1r)


def _tc_out(s2_p, g2, dinv16, W2, b2r):
    def body(s_ref, g_ref, dinv_ref, w_ref, b_ref, o_ref):
        t = dinv_ref[...] * (s_ref[0] + s_ref[1] + g_ref[...])
        o_ref[...] = jnp.dot(t, w_ref[...],
                             preferred_element_type=jnp.float32) + b_ref[...]

    return pl.pallas_call(
        body,
        grid=(NPAD // 1280,),
        in_specs=[
            pl.BlockSpec((NC, 1280, H1), lambda i: (0, i, 0)),
            pl.BlockSpec((1280, H1), lambda i: (i, 0)),
            pl.BlockSpec((1280, H1), lambda i: (i, 0)),
            pl.BlockSpec((H1, H2), lambda i: (0, 0)),
            pl.BlockSpec((1, H2), lambda i: (0, 0)),
        ],
        out_specs=pl.BlockSpec((1280, H2), lambda i: (i, 0)),
        out_shape=jax.ShapeDtypeStruct((NPAD, H2), jnp.float32),
    )(s2_p, g2, dinv16, W2, b2r)


def kernel(x, edge_index, W1, b1, W2, b2):
    src = edge_index[0]
    dst = edge_index[1]
    pad = (jnp.arange(EPAD - E, dtype=jnp.int32) % (NPAD - N)) + N
    srcp = jnp.concatenate([src, pad]).reshape(NROWS, 128)
    dstp = jnp.concatenate([dst, pad]).reshape(NROWS, 128)
    x_pad = jnp.pad(x, ((0, NPAD - N), (0, 0)))
    b1r = b1.reshape(1, H1)
    b2r = b2.reshape(1, H2)

    cnt_p = _sc_degree(dstp)
    h1 = _tc_matmul1(x_pad, W1)
    dinv16, g1 = _tc_scale(cnt_p, h1)
    s1_p = _sc_aggregate(srcp, dstp, g1)
    g2 = _tc_relu_scale(s1_p, g1, dinv16, b1r)
    s2_p = _sc_aggregate(srcp, dstp, g2)
    out = _tc_out(s2_p, g2, dinv16, W2, b2r)
    return out[:N]
